# baseline (speedup 1.0000x reference)
import functools

import jax
import jax.numpy as jnp
from jax import lax
from jax.experimental import pallas as pl
from jax.experimental.pallas import tpu as pltpu
from jax.experimental.pallas import tpu_sc as plsc

N_VOCAB = 1000000
N_EMBED = 64
N_TOKENS = 16384 * 50

NC = 2
NS = 16
NW = NC * NS

G = 128
N_GROUPS = N_TOKENS // (NW * G)
K = 5
N_CHUNKS = N_GROUPS // K

_mesh = plsc.VectorSubcoreMesh(
    core_axis_name="c", subcore_axis_name="s", num_cores=NC, num_subcores=NS
)

BLK_V = 2000


def _repack_body(w_ref, o_ref):
    w3 = w_ref[...].reshape(BLK_V // 2, 2, N_EMBED)
    o_ref[...] = jnp.concatenate([w3[:, 0, :], w3[:, 1, :]], axis=1)


_repack = pl.pallas_call(
    _repack_body,
    grid=(N_VOCAB // BLK_V,),
    in_specs=[pl.BlockSpec((BLK_V, N_EMBED), lambda i: (i, 0))],
    out_specs=pl.BlockSpec((BLK_V // 2, 128), lambda i: (i, 0)),
    out_shape=jax.ShapeDtypeStruct((N_VOCAB // 2, 128), jnp.float32),
)

TB = 128
FPT = N_EMBED * 50 // 128


def _unpack_body(f_ref, o_ref):
    f3 = f_ref[...].reshape(TB, FPT, 128)
    even = f3[:, :, 0:N_EMBED]
    odd = f3[:, :, N_EMBED:128]
    o_ref[...] = jnp.stack([even, odd], axis=2).reshape(TB, 50, N_EMBED)


_unpack = pl.pallas_call(
    _unpack_body,
    grid=(16384 // TB,),
    in_specs=[pl.BlockSpec((TB * FPT, 128), lambda i: (i, 0))],
    out_specs=pl.BlockSpec((TB, 50, N_EMBED), lambda i: (i, 0, 0)),
    out_shape=jax.ShapeDtypeStruct((16384, 50, N_EMBED), jnp.float32),
)


@functools.partial(
    pl.kernel,
    mesh=_mesh,
    compiler_params=pltpu.CompilerParams(use_tc_tiling_on_sc=False),
    out_type=jax.ShapeDtypeStruct((N_TOKENS // G, G, N_EMBED), jnp.float32),
    scratch_types=[
        pltpu.VMEM((N_GROUPS, G), jnp.int32),
        pltpu.VMEM((K, G, N_EMBED), jnp.float32),
        pltpu.VMEM((K, G, N_EMBED), jnp.float32),
        pltpu.SemaphoreType.DMA,
        pltpu.SemaphoreType.DMA,
        pltpu.SemaphoreType.DMA,
        pltpu.SemaphoreType.DMA,
    ],
)
def _emb_lookup(idx_hbm, table_hbm, out_hbm, idx_all, rows0, rows1,
                semg0, semg1, sems0, sems1):
    wid = lax.axis_index("s") * NC + lax.axis_index("c")
    gbase = wid * N_GROUPS

    pltpu.sync_copy(idx_hbm.at[pl.ds(gbase, N_GROUPS)], idx_all)

    rows = (rows0, rows1)
    semg = (semg0, semg1)
    sems = (sems0, sems1)

    def fire_gathers(c, b):
        for j in range(K):
            pltpu.async_copy(
                table_hbm.at[idx_all.at[c * K + j]], rows[b].at[j], semg[b]
            )

    def drain_gathers(b):
        pltpu.make_async_copy(out_hbm.at[pl.ds(0, K)], rows[b], semg[b]).wait()

    def fire_store(c, b):
        pltpu.async_copy(rows[b], out_hbm.at[pl.ds(gbase + c * K, K)], sems[b])

    def drain_store(b):
        pltpu.make_async_copy(rows[b], out_hbm.at[pl.ds(0, K)], sems[b]).wait()

    fire_gathers(0, 0)

    def pair_body(p, carry):
        c0 = 2 * p

        @pl.when(p >= 1)
        def _():
            drain_store(1)

        fire_gathers(c0 + 1, 1)
        drain_gathers(0)
        fire_store(c0, 0)

        @pl.when(p < N_CHUNKS // 2 - 1)
        def _():
            drain_store(0)
            fire_gathers(c0 + 2, 0)

        drain_gathers(1)
        fire_store(c0 + 1, 1)
        return carry

    lax.fori_loop(0, N_CHUNKS // 2, pair_body, 0)

    drain_store(0)
    drain_store(1)


def kernel(x, weight):
    wlin = _repack(weight)
    table = wlin.reshape(N_VOCAB, N_EMBED)
    idx2d = x.reshape(N_TOKENS // G, G).astype(jnp.int32)
    flat = _emb_lookup(idx2d, table)
    flat2 = flat.reshape(N_TOKENS * N_EMBED // 128, 128)
    return _unpack(flat2)


# speedup vs baseline: 1.3208x; 1.3208x over previous
import functools

import jax
import jax.numpy as jnp
from jax import lax
from jax.experimental import pallas as pl
from jax.experimental.pallas import tpu as pltpu
from jax.experimental.pallas import tpu_sc as plsc

N_VOCAB = 1000000
N_EMBED = 64
N_TOKENS = 16384 * 50

NC = 2
NS = 16
NW = NC * NS

G = 128
N_GROUPS = N_TOKENS // (NW * G)
K = 5
N_CHUNKS = N_GROUPS // K

_mesh = plsc.VectorSubcoreMesh(
    core_axis_name="c", subcore_axis_name="s", num_cores=NC, num_subcores=NS
)


def _make_emb(total_groups):
    n_groups = total_groups // NW
    n_chunks = n_groups // K

    @functools.partial(
        pl.kernel,
        mesh=_mesh,
        compiler_params=pltpu.CompilerParams(use_tc_tiling_on_sc=False),
        out_type=jax.ShapeDtypeStruct((total_groups, G, N_EMBED), jnp.float32),
        scratch_types=[
            pltpu.VMEM((n_groups, G), jnp.int32),
            pltpu.VMEM((K, G, N_EMBED), jnp.float32),
            pltpu.VMEM((K, G, N_EMBED), jnp.float32),
            pltpu.SemaphoreType.DMA,
            pltpu.SemaphoreType.DMA,
            pltpu.SemaphoreType.DMA,
            pltpu.SemaphoreType.DMA,
        ],
    )
    def _emb_lookup(idx_hbm, table_hbm, out_hbm, idx_all, rows0, rows1,
                    semg0, semg1, sems0, sems1):
        wid = lax.axis_index("s") * NC + lax.axis_index("c")
        gbase = wid * n_groups

        pltpu.sync_copy(idx_hbm.at[pl.ds(gbase, n_groups)], idx_all)

        rows = (rows0, rows1)
        semg = (semg0, semg1)
        sems = (sems0, sems1)

        def fire_gathers(c, b):
            for j in range(K):
                pltpu.async_copy(
                    table_hbm.at[idx_all.at[c * K + j]], rows[b].at[j], semg[b]
                )

        def drain_gathers(b):
            pltpu.make_async_copy(
                out_hbm.at[pl.ds(0, K)], rows[b], semg[b]).wait()

        def fire_store(c, b):
            pltpu.async_copy(
                rows[b], out_hbm.at[pl.ds(gbase + c * K, K)], sems[b])

        def drain_store(b):
            pltpu.make_async_copy(
                rows[b], out_hbm.at[pl.ds(0, K)], sems[b]).wait()

        fire_gathers(0, 0)

        def pair_body(p, carry):
            c0 = 2 * p

            @pl.when(p >= 1)
            def _():
                drain_store(1)

            fire_gathers(c0 + 1, 1)
            drain_gathers(0)
            fire_store(c0, 0)

            @pl.when(p < n_chunks // 2 - 1)
            def _():
                drain_store(0)
                fire_gathers(c0 + 2, 0)

            drain_gathers(1)
            fire_store(c0 + 1, 1)
            return carry

        lax.fori_loop(0, n_chunks // 2, pair_body, 0)

        drain_store(0)
        drain_store(1)

    return _emb_lookup


HALVES = 2
_PW = 50 // HALVES
_HG = N_TOKENS // HALVES // G
_emb_half = _make_emb(_HG)


def kernel(x, weight):
    outs = []
    for h in range(HALVES):
        xh = x[:, h * _PW:(h + 1) * _PW].reshape(_HG, G).astype(jnp.int32)
        oh = _emb_half(xh, weight)
        outs.append(oh.reshape(x.shape[0], _PW, weight.shape[1]))
    return jnp.concatenate(outs, axis=1)


# speedup vs baseline: 1.4373x; 1.0882x over previous
import functools

import jax
import jax.numpy as jnp
from jax import lax
from jax.experimental import pallas as pl
from jax.experimental.pallas import tpu as pltpu
from jax.experimental.pallas import tpu_sc as plsc

N_VOCAB = 1000000
N_EMBED = 64
N_TOKENS = 16384 * 50

NC = 2
NS = 16
NW = NC * NS

G = 128
N_GROUPS = N_TOKENS // (NW * G)
K = 5
N_CHUNKS = N_GROUPS // K

_mesh = plsc.VectorSubcoreMesh(
    core_axis_name="c", subcore_axis_name="s", num_cores=NC, num_subcores=NS
)


def _make_emb(total_groups):
    n_groups = total_groups // NW
    n_chunks = n_groups // K

    @functools.partial(
        pl.kernel,
        mesh=_mesh,
        compiler_params=pltpu.CompilerParams(use_tc_tiling_on_sc=False),
        out_type=jax.ShapeDtypeStruct((total_groups, G, N_EMBED), jnp.float32),
        scratch_types=[
            pltpu.VMEM((n_groups, G), jnp.int32),
            pltpu.VMEM((K, G, N_EMBED), jnp.float32),
            pltpu.VMEM((K, G, N_EMBED), jnp.float32),
            pltpu.SemaphoreType.DMA,
            pltpu.SemaphoreType.DMA,
            pltpu.SemaphoreType.DMA,
            pltpu.SemaphoreType.DMA,
        ],
    )
    def _emb_lookup(idx_hbm, table_hbm, out_hbm, idx_all, rows0, rows1,
                    semg0, semg1, sems0, sems1):
        wid = lax.axis_index("s") * NC + lax.axis_index("c")
        gbase = wid * n_groups

        pltpu.sync_copy(idx_hbm.at[pl.ds(gbase, n_groups)], idx_all)

        rows = (rows0, rows1)
        semg = (semg0, semg1)
        sems = (sems0, sems1)

        def fire_gathers(c, b):
            for j in range(K):
                pltpu.async_copy(
                    table_hbm.at[idx_all.at[c * K + j]], rows[b].at[j], semg[b]
                )

        def drain_gathers(b):
            pltpu.make_async_copy(
                out_hbm.at[pl.ds(0, K)], rows[b], semg[b]).wait()

        def fire_store(c, b):
            pltpu.async_copy(
                rows[b], out_hbm.at[pl.ds(gbase + c * K, K)], sems[b])

        def drain_store(b):
            pltpu.make_async_copy(
                rows[b], out_hbm.at[pl.ds(0, K)], sems[b]).wait()

        fire_gathers(0, 0)

        def pair_body(p, carry):
            c0 = 2 * p

            @pl.when(p >= 1)
            def _():
                drain_store(1)

            fire_gathers(c0 + 1, 1)
            drain_gathers(0)
            fire_store(c0, 0)

            @pl.when(p < n_chunks // 2 - 1)
            def _():
                drain_store(0)
                fire_gathers(c0 + 2, 0)

            drain_gathers(1)
            fire_store(c0 + 1, 1)
            return carry

        lax.fori_loop(0, n_chunks // 2, pair_body, 0)

        drain_store(0)
        drain_store(1)

    return _emb_lookup


_emb = _make_emb(N_TOKENS // G)


def kernel(x, weight):
    idx2d = x.reshape(N_TOKENS // G, G).astype(jnp.int32)
    out = _emb(idx2d, weight)
    return out.reshape(x.shape + (weight.shape[1],))
